# baseline (speedup 1.0000x reference)
import functools

import jax
import jax.numpy as jnp
import numpy as np
from jax import lax
from jax.experimental import pallas as pl
from jax.experimental.pallas import tpu as pltpu

K, T, D, H = 128, 64, 128, 256
TEMP = 1.0
EPS = 1e-12
LOG2 = float(np.log(2.0))

KB = 16
GRID = K // KB

NEG_INF = np.float32(-np.inf)


_GUMBEL = np.asarray(
    jax.random.gumbel(jax.random.key(42), (K, T * T), jnp.float32)
).reshape(K, T, T)


def _body(emb_ref, gum_ref, w1_ref, b1_ref, w2_ref, b2_ref,
          idx1_ref, idx2_ref, br1_ref, br2_ref, out_ref, logv_ref):
    rr = lax.broadcasted_iota(jnp.int32, (T, T), 0)
    cc = lax.broadcasted_iota(jnp.int32, (T, T), 1)
    diag = rr == cc
    flat_idx = rr * T + cc

    krows_cat = lax.broadcasted_iota(jnp.int32, (KB, 2 * D), 0)
    krows_1 = lax.broadcasted_iota(jnp.int32, (KB, 1), 0)

    def per_particle(i, carry):
        cat_c, i1_c, i2_c, logv_c = carry
        e = emb_ref[i]
        gram = lax.dot_general(e, e, (((1,), (1,)), ((), ())),
                               preferred_element_type=jnp.float32,
                               precision=lax.Precision.HIGHEST)
        diag_vals = jnp.where(diag, gram, 0.0)
        n_col = jnp.sum(diag_vals, axis=1, keepdims=True)
        n_row = jnp.sum(diag_vals, axis=0, keepdims=True)
        sq = jnp.maximum(n_col + n_row - 2.0 * gram, 0.0)
        dist = jnp.sqrt(sq + EPS)
        logits = jnp.where(diag, NEG_INF, -dist / TEMP)
        scores = logits + gum_ref[i]
        m = jnp.max(scores)
        samp = jnp.min(jnp.where(scores == m, flat_idx, T * T))
        i1 = samp // T
        i2 = samp % T
        chosen = jnp.sum(jnp.where(flat_idx == samp, logits, 0.0))
        ml = jnp.max(logits)
        lse = ml + jnp.log(jnp.sum(jnp.exp(logits - ml)))
        rows = lax.broadcasted_iota(jnp.int32, (T, D), 0)
        c1 = jnp.sum(jnp.where(rows == i1, e, 0.0), axis=0, keepdims=True)
        c2 = jnp.sum(jnp.where(rows == i2, e, 0.0), axis=0, keepdims=True)
        cat_row = jnp.concatenate([c1, c2], axis=1)
        cat_c = jnp.where(krows_cat == i, cat_row, cat_c)
        i1_c = jnp.where(krows_1 == i, i1, i1_c)
        i2_c = jnp.where(krows_1 == i, i2, i2_c)
        logv_c = jnp.where(krows_1 == i, chosen + LOG2 - lse, logv_c)
        return cat_c, i1_c, i2_c, logv_c

    cat, i1s, i2s, logvs = lax.fori_loop(
        0, KB, per_particle,
        (jnp.zeros((KB, 2 * D), jnp.float32),
         jnp.zeros((KB, 1), jnp.int32),
         jnp.zeros((KB, 1), jnp.int32),
         jnp.zeros((KB, 1), jnp.float32)))
    idx1_ref[:] = i1s
    idx2_ref[:] = i2s
    logv_ref[:] = logvs
    h = lax.dot_general(cat, w1_ref[:], (((1,), (0,)), ((), ())),
                        preferred_element_type=jnp.float32) + b1_ref[:]
    h = jnp.maximum(h, 0.0)
    out = lax.dot_general(h, w2_ref[:], (((1,), (0,)), ((), ())),
                          preferred_element_type=jnp.float32) + b2_ref[:]
    out_ref[:] = out
    c1b = cat[:, 0:D]
    c2b = cat[:, D:2 * D]
    br1_ref[:] = jnp.sqrt(jnp.sum((c1b - out) ** 2, axis=1, keepdims=True)
                          + EPS)
    br2_ref[:] = jnp.sqrt(jnp.sum((c2b - out) ** 2, axis=1, keepdims=True)
                          + EPS)


def kernel(N, leaf_counts_Kxt, embeddings_KxtxD, W1, b1, W2, b2):
    gum = jnp.asarray(_GUMBEL)
    idx1, idx2, br1, br2, emb_out, logv = pl.pallas_call(
        _body,
        grid=(GRID,),
        in_specs=[
            pl.BlockSpec((KB, T, D), lambda i: (i, 0, 0)),
            pl.BlockSpec((KB, T, T), lambda i: (i, 0, 0)),
            pl.BlockSpec((2 * D, H), lambda i: (0, 0)),
            pl.BlockSpec((1, H), lambda i: (0, 0)),
            pl.BlockSpec((H, D), lambda i: (0, 0)),
            pl.BlockSpec((1, D), lambda i: (0, 0)),
        ],
        out_specs=[
            pl.BlockSpec((KB, 1), lambda i: (i, 0)),
            pl.BlockSpec((KB, 1), lambda i: (i, 0)),
            pl.BlockSpec((KB, 1), lambda i: (i, 0)),
            pl.BlockSpec((KB, 1), lambda i: (i, 0)),
            pl.BlockSpec((KB, D), lambda i: (i, 0)),
            pl.BlockSpec((KB, 1), lambda i: (i, 0)),
        ],
        out_shape=[
            jax.ShapeDtypeStruct((K, 1), jnp.int32),
            jax.ShapeDtypeStruct((K, 1), jnp.int32),
            jax.ShapeDtypeStruct((K, 1), jnp.float32),
            jax.ShapeDtypeStruct((K, 1), jnp.float32),
            jax.ShapeDtypeStruct((K, D), jnp.float32),
            jax.ShapeDtypeStruct((K, 1), jnp.float32),
        ],
    )(embeddings_KxtxD, gum, W1, b1.reshape(1, H), W2, b2.reshape(1, D))
    return (idx1, idx2, br1, br2, emb_out[:, None, :], logv)


# speedup vs baseline: 2.1454x; 2.1454x over previous
import jax
import jax.numpy as jnp
import numpy as np
from jax import lax
from jax.experimental import pallas as pl
from jax.experimental.pallas import tpu as pltpu

K, T, D, H = 128, 64, 128, 256
TEMP = 1.0
EPS = 1e-12
LOG2 = float(np.log(2.0))

KB = 16
GRID = K // KB

NEG_INF = np.float32(-np.inf)

_GUMBEL = np.asarray(
    jax.random.gumbel(jax.random.key(42), (K, T * T), jnp.float32)
).reshape(K, T, T)


def _body(emb_ref, gum_ref, w1_ref, b1_ref, w2_ref, b2_ref,
          idx1_ref, idx2_ref, br1_ref, br2_ref, out_ref, logv_ref,
          gram_s, ncol_s, nrow_s):
    diag2 = (lax.broadcasted_iota(jnp.int32, (T, T), 0)
             == lax.broadcasted_iota(jnp.int32, (T, T), 1))

    def gram_step(i, carry):
        e = emb_ref[i]
        g = lax.dot_general(e, e, (((1,), (1,)), ((), ())),
                            preferred_element_type=jnp.float32,
                            precision=lax.Precision.HIGHEST)
        dv = jnp.where(diag2, g, 0.0)
        gram_s[i] = g
        ncol_s[i] = jnp.sum(dv, axis=1, keepdims=True)
        nrow_s[i] = jnp.sum(dv, axis=0, keepdims=True)
        return carry

    lax.fori_loop(0, KB, gram_step, 0)

    g3 = gram_s[:]
    sq3 = jnp.maximum(ncol_s[:] + nrow_s[:] - 2.0 * g3, 0.0)
    dist3 = jnp.sqrt(sq3 + EPS)
    rr3 = lax.broadcasted_iota(jnp.int32, (KB, T, T), 1)
    cc3 = lax.broadcasted_iota(jnp.int32, (KB, T, T), 2)
    diag3 = rr3 == cc3
    flat3 = rr3 * T + cc3
    logits3 = jnp.where(diag3, NEG_INF, -dist3 / TEMP)
    scores3 = logits3 + gum_ref[:]

    m = jnp.max(jnp.max(scores3, axis=2), axis=1, keepdims=True)
    cand3 = jnp.where(scores3 == m[:, :, None], flat3, T * T)
    samp = jnp.min(jnp.min(cand3, axis=2), axis=1, keepdims=True)
    i1s = samp // T
    i2s = samp % T
    hit3 = jnp.where(flat3 == samp[:, :, None], logits3, 0.0)
    chosen = jnp.sum(jnp.sum(hit3, axis=2), axis=1, keepdims=True)
    ml = jnp.max(jnp.max(logits3, axis=2), axis=1, keepdims=True)
    s = jnp.sum(jnp.sum(jnp.exp(logits3 - ml[:, :, None]), axis=2),
                axis=1, keepdims=True)
    lse = ml + jnp.log(s)
    idx1_ref[:] = i1s
    idx2_ref[:] = i2s
    logv_ref[:] = chosen + LOG2 - lse

    krows = lax.broadcasted_iota(jnp.int32, (KB, 1), 0)
    lanes = lax.broadcasted_iota(jnp.int32, (KB, KB * T), 1)
    m1 = (lanes == krows * T + i1s).astype(jnp.float32)
    m2 = (lanes == krows * T + i2s).astype(jnp.float32)
    e_flat = emb_ref[:].reshape(KB * T, D)
    c1 = lax.dot_general(m1, e_flat, (((1,), (0,)), ((), ())),
                         preferred_element_type=jnp.float32,
                         precision=lax.Precision.HIGHEST)
    c2 = lax.dot_general(m2, e_flat, (((1,), (0,)), ((), ())),
                         preferred_element_type=jnp.float32,
                         precision=lax.Precision.HIGHEST)
    cat = jnp.concatenate([c1, c2], axis=1)

    h = lax.dot_general(cat, w1_ref[:], (((1,), (0,)), ((), ())),
                        preferred_element_type=jnp.float32) + b1_ref[:]
    h = jnp.maximum(h, 0.0)
    out = lax.dot_general(h, w2_ref[:], (((1,), (0,)), ((), ())),
                          preferred_element_type=jnp.float32) + b2_ref[:]
    out_ref[:] = out
    br1_ref[:] = jnp.sqrt(jnp.sum((c1 - out) ** 2, axis=1, keepdims=True)
                          + EPS)
    br2_ref[:] = jnp.sqrt(jnp.sum((c2 - out) ** 2, axis=1, keepdims=True)
                          + EPS)


def kernel(N, leaf_counts_Kxt, embeddings_KxtxD, W1, b1, W2, b2):
    gum = jnp.asarray(_GUMBEL)
    idx1, idx2, br1, br2, emb_out, logv = pl.pallas_call(
        _body,
        grid=(GRID,),
        in_specs=[
            pl.BlockSpec((KB, T, D), lambda i: (i, 0, 0)),
            pl.BlockSpec((KB, T, T), lambda i: (i, 0, 0)),
            pl.BlockSpec((2 * D, H), lambda i: (0, 0)),
            pl.BlockSpec((1, H), lambda i: (0, 0)),
            pl.BlockSpec((H, D), lambda i: (0, 0)),
            pl.BlockSpec((1, D), lambda i: (0, 0)),
        ],
        out_specs=[
            pl.BlockSpec((KB, 1), lambda i: (i, 0)),
            pl.BlockSpec((KB, 1), lambda i: (i, 0)),
            pl.BlockSpec((KB, 1), lambda i: (i, 0)),
            pl.BlockSpec((KB, 1), lambda i: (i, 0)),
            pl.BlockSpec((KB, D), lambda i: (i, 0)),
            pl.BlockSpec((KB, 1), lambda i: (i, 0)),
        ],
        out_shape=[
            jax.ShapeDtypeStruct((K, 1), jnp.int32),
            jax.ShapeDtypeStruct((K, 1), jnp.int32),
            jax.ShapeDtypeStruct((K, 1), jnp.float32),
            jax.ShapeDtypeStruct((K, 1), jnp.float32),
            jax.ShapeDtypeStruct((K, D), jnp.float32),
            jax.ShapeDtypeStruct((K, 1), jnp.float32),
        ],
        scratch_shapes=[
            pltpu.VMEM((KB, T, T), jnp.float32),
            pltpu.VMEM((KB, T, 1), jnp.float32),
            pltpu.VMEM((KB, 1, T), jnp.float32),
        ],
    )(embeddings_KxtxD, gum, W1, b1.reshape(1, H), W2, b2.reshape(1, D))
    return (idx1, idx2, br1, br2, emb_out[:, None, :], logv)


# speedup vs baseline: 3.8948x; 1.8154x over previous
import jax
import jax.numpy as jnp
import numpy as np
from jax import lax
from jax.experimental import pallas as pl
from jax.experimental.pallas import tpu as pltpu

K, T, D, H = 128, 64, 128, 256
TEMP = 1.0
EPS = 1e-12
LOG2 = float(np.log(2.0))

KB = 16
GRID = K // KB

NEG_INF = np.float32(-np.inf)

_GUMBEL = np.asarray(
    jax.random.gumbel(jax.random.key(42), (K, T * T), jnp.float32)
).reshape(K, T, T)


def _body(emb_ref, gum_ref, w1_ref, b1_ref, w2_ref, b2_ref,
          idx1_ref, idx2_ref, br1_ref, br2_ref, out_ref, logv_ref):
    diag2 = (lax.broadcasted_iota(jnp.int32, (T, T), 0)
             == lax.broadcasted_iota(jnp.int32, (T, T), 1))

    grams, ncols, nrows = [], [], []
    for i in range(KB):
        e = emb_ref[i]
        g = lax.dot_general(e, e, (((1,), (1,)), ((), ())),
                            preferred_element_type=jnp.float32,
                            precision=lax.Precision.HIGHEST)
        dv = jnp.where(diag2, g, 0.0)
        grams.append(g[None])
        ncols.append(jnp.sum(dv, axis=1, keepdims=True)[None])
        nrows.append(jnp.sum(dv, axis=0, keepdims=True)[None])

    g3 = jnp.concatenate(grams, axis=0)
    ncol3 = jnp.concatenate(ncols, axis=0)
    nrow3 = jnp.concatenate(nrows, axis=0)
    sq3 = jnp.maximum(ncol3 + nrow3 - 2.0 * g3, 0.0)
    dist3 = jnp.sqrt(sq3 + EPS)
    rr3 = lax.broadcasted_iota(jnp.int32, (KB, T, T), 1)
    cc3 = lax.broadcasted_iota(jnp.int32, (KB, T, T), 2)
    diag3 = rr3 == cc3
    flat3 = rr3 * T + cc3
    logits3 = jnp.where(diag3, NEG_INF, -dist3 / TEMP)
    scores3 = logits3 + gum_ref[:]

    m = jnp.max(jnp.max(scores3, axis=2), axis=1, keepdims=True)
    cand3 = jnp.where(scores3 == m[:, :, None], flat3, T * T)
    samp = jnp.min(jnp.min(cand3, axis=2), axis=1, keepdims=True)
    i1s = samp // T
    i2s = samp % T
    hit3 = jnp.where(flat3 == samp[:, :, None], logits3, 0.0)
    chosen = jnp.sum(jnp.sum(hit3, axis=2), axis=1, keepdims=True)
    ml = jnp.max(jnp.max(logits3, axis=2), axis=1, keepdims=True)
    s = jnp.sum(jnp.sum(jnp.exp(logits3 - ml[:, :, None]), axis=2),
                axis=1, keepdims=True)
    lse = ml + jnp.log(s)
    idx1_ref[:] = i1s
    idx2_ref[:] = i2s
    logv_ref[:] = chosen + LOG2 - lse

    krows = lax.broadcasted_iota(jnp.int32, (KB, 1), 0)
    lanes = lax.broadcasted_iota(jnp.int32, (KB, KB * T), 1)
    m1 = (lanes == krows * T + i1s).astype(jnp.float32)
    m2 = (lanes == krows * T + i2s).astype(jnp.float32)
    e_flat = emb_ref[:].reshape(KB * T, D)
    c1 = lax.dot_general(m1, e_flat, (((1,), (0,)), ((), ())),
                         preferred_element_type=jnp.float32,
                         precision=lax.Precision.HIGHEST)
    c2 = lax.dot_general(m2, e_flat, (((1,), (0,)), ((), ())),
                         preferred_element_type=jnp.float32,
                         precision=lax.Precision.HIGHEST)
    cat = jnp.concatenate([c1, c2], axis=1)

    h = lax.dot_general(cat, w1_ref[:], (((1,), (0,)), ((), ())),
                        preferred_element_type=jnp.float32) + b1_ref[:]
    h = jnp.maximum(h, 0.0)
    out = lax.dot_general(h, w2_ref[:], (((1,), (0,)), ((), ())),
                          preferred_element_type=jnp.float32) + b2_ref[:]
    out_ref[:] = out
    br1_ref[:] = jnp.sqrt(jnp.sum((c1 - out) ** 2, axis=1, keepdims=True)
                          + EPS)
    br2_ref[:] = jnp.sqrt(jnp.sum((c2 - out) ** 2, axis=1, keepdims=True)
                          + EPS)


def kernel(N, leaf_counts_Kxt, embeddings_KxtxD, W1, b1, W2, b2):
    gum = jnp.asarray(_GUMBEL)
    idx1, idx2, br1, br2, emb_out, logv = pl.pallas_call(
        _body,
        grid=(GRID,),
        in_specs=[
            pl.BlockSpec((KB, T, D), lambda i: (i, 0, 0)),
            pl.BlockSpec((KB, T, T), lambda i: (i, 0, 0)),
            pl.BlockSpec((2 * D, H), lambda i: (0, 0)),
            pl.BlockSpec((1, H), lambda i: (0, 0)),
            pl.BlockSpec((H, D), lambda i: (0, 0)),
            pl.BlockSpec((1, D), lambda i: (0, 0)),
        ],
        out_specs=[
            pl.BlockSpec((KB, 1), lambda i: (i, 0)),
            pl.BlockSpec((KB, 1), lambda i: (i, 0)),
            pl.BlockSpec((KB, 1), lambda i: (i, 0)),
            pl.BlockSpec((KB, 1), lambda i: (i, 0)),
            pl.BlockSpec((KB, D), lambda i: (i, 0)),
            pl.BlockSpec((KB, 1), lambda i: (i, 0)),
        ],
        out_shape=[
            jax.ShapeDtypeStruct((K, 1), jnp.int32),
            jax.ShapeDtypeStruct((K, 1), jnp.int32),
            jax.ShapeDtypeStruct((K, 1), jnp.float32),
            jax.ShapeDtypeStruct((K, 1), jnp.float32),
            jax.ShapeDtypeStruct((K, D), jnp.float32),
            jax.ShapeDtypeStruct((K, 1), jnp.float32),
        ],
    )(embeddings_KxtxD, gum, W1, b1.reshape(1, H), W2, b2.reshape(1, D))
    return (idx1, idx2, br1, br2, emb_out[:, None, :], logv)


# speedup vs baseline: 4.3835x; 1.1255x over previous
import jax
import jax.numpy as jnp
import numpy as np
from jax import lax
from jax.experimental import pallas as pl
from jax.experimental.pallas import tpu as pltpu

K, T, D, H = 128, 64, 128, 256
TEMP = 1.0
EPS = 1e-12
LOG2 = float(np.log(2.0))

KB = 32
GRID = K // KB

NEG_INF = np.float32(-np.inf)

_GUMBEL = np.asarray(
    jax.random.gumbel(jax.random.key(42), (K, T * T), jnp.float32)
).reshape(K, T, T)


def _body(emb_ref, gum_ref, w1_ref, b1_ref, w2_ref, b2_ref,
          idx1_ref, idx2_ref, br1_ref, br2_ref, out_ref, logv_ref):
    diag2 = (lax.broadcasted_iota(jnp.int32, (T, T), 0)
             == lax.broadcasted_iota(jnp.int32, (T, T), 1))

    grams, ncols, nrows = [], [], []
    for i in range(KB):
        e = emb_ref[i]
        g = lax.dot_general(e, e, (((1,), (1,)), ((), ())),
                            preferred_element_type=jnp.float32,
                            precision=lax.Precision.HIGHEST)
        dv = jnp.where(diag2, g, 0.0)
        grams.append(g[None])
        ncols.append(jnp.sum(dv, axis=1, keepdims=True)[None])
        nrows.append(jnp.sum(dv, axis=0, keepdims=True)[None])

    g3 = jnp.concatenate(grams, axis=0)
    ncol3 = jnp.concatenate(ncols, axis=0)
    nrow3 = jnp.concatenate(nrows, axis=0)
    sq3 = jnp.maximum(ncol3 + nrow3 - 2.0 * g3, 0.0)
    dist3 = jnp.sqrt(sq3 + EPS)
    rr3 = lax.broadcasted_iota(jnp.int32, (KB, T, T), 1)
    cc3 = lax.broadcasted_iota(jnp.int32, (KB, T, T), 2)
    diag3 = rr3 == cc3
    flat3 = rr3 * T + cc3
    logits3 = jnp.where(diag3, NEG_INF, -dist3 / TEMP)
    scores3 = logits3 + gum_ref[:]

    m = jnp.max(jnp.max(scores3, axis=2), axis=1, keepdims=True)
    cand3 = jnp.where(scores3 == m[:, :, None], flat3, T * T)
    samp = jnp.min(jnp.min(cand3, axis=2), axis=1, keepdims=True)
    i1s = samp // T
    i2s = samp % T
    hit3 = jnp.where(flat3 == samp[:, :, None], logits3, 0.0)
    chosen = jnp.sum(jnp.sum(hit3, axis=2), axis=1, keepdims=True)
    ml = jnp.max(jnp.max(logits3, axis=2), axis=1, keepdims=True)
    s = jnp.sum(jnp.sum(jnp.exp(logits3 - ml[:, :, None]), axis=2),
                axis=1, keepdims=True)
    lse = ml + jnp.log(s)
    idx1_ref[:] = i1s
    idx2_ref[:] = i2s
    logv_ref[:] = chosen + LOG2 - lse

    krows = lax.broadcasted_iota(jnp.int32, (KB, 1), 0)
    lanes = lax.broadcasted_iota(jnp.int32, (KB, KB * T), 1)
    m1 = (lanes == krows * T + i1s).astype(jnp.float32)
    m2 = (lanes == krows * T + i2s).astype(jnp.float32)
    e_flat = emb_ref[:].reshape(KB * T, D)
    c1 = lax.dot_general(m1, e_flat, (((1,), (0,)), ((), ())),
                         preferred_element_type=jnp.float32,
                         precision=lax.Precision.HIGHEST)
    c2 = lax.dot_general(m2, e_flat, (((1,), (0,)), ((), ())),
                         preferred_element_type=jnp.float32,
                         precision=lax.Precision.HIGHEST)
    cat = jnp.concatenate([c1, c2], axis=1)

    h = lax.dot_general(cat, w1_ref[:], (((1,), (0,)), ((), ())),
                        preferred_element_type=jnp.float32) + b1_ref[:]
    h = jnp.maximum(h, 0.0)
    out = lax.dot_general(h, w2_ref[:], (((1,), (0,)), ((), ())),
                          preferred_element_type=jnp.float32) + b2_ref[:]
    out_ref[:] = out
    br1_ref[:] = jnp.sqrt(jnp.sum((c1 - out) ** 2, axis=1, keepdims=True)
                          + EPS)
    br2_ref[:] = jnp.sqrt(jnp.sum((c2 - out) ** 2, axis=1, keepdims=True)
                          + EPS)


def kernel(N, leaf_counts_Kxt, embeddings_KxtxD, W1, b1, W2, b2):
    gum = jnp.asarray(_GUMBEL)
    idx1, idx2, br1, br2, emb_out, logv = pl.pallas_call(
        _body,
        grid=(GRID,),
        in_specs=[
            pl.BlockSpec((KB, T, D), lambda i: (i, 0, 0)),
            pl.BlockSpec((KB, T, T), lambda i: (i, 0, 0)),
            pl.BlockSpec((2 * D, H), lambda i: (0, 0)),
            pl.BlockSpec((1, H), lambda i: (0, 0)),
            pl.BlockSpec((H, D), lambda i: (0, 0)),
            pl.BlockSpec((1, D), lambda i: (0, 0)),
        ],
        out_specs=[
            pl.BlockSpec((KB, 1), lambda i: (i, 0)),
            pl.BlockSpec((KB, 1), lambda i: (i, 0)),
            pl.BlockSpec((KB, 1), lambda i: (i, 0)),
            pl.BlockSpec((KB, 1), lambda i: (i, 0)),
            pl.BlockSpec((KB, D), lambda i: (i, 0)),
            pl.BlockSpec((KB, 1), lambda i: (i, 0)),
        ],
        out_shape=[
            jax.ShapeDtypeStruct((K, 1), jnp.int32),
            jax.ShapeDtypeStruct((K, 1), jnp.int32),
            jax.ShapeDtypeStruct((K, 1), jnp.float32),
            jax.ShapeDtypeStruct((K, 1), jnp.float32),
            jax.ShapeDtypeStruct((K, D), jnp.float32),
            jax.ShapeDtypeStruct((K, 1), jnp.float32),
        ],
    )(embeddings_KxtxD, gum, W1, b1.reshape(1, H), W2, b2.reshape(1, D))
    return (idx1, idx2, br1, br2, emb_out[:, None, :], logv)


# speedup vs baseline: 5.2481x; 1.1972x over previous
import jax
import jax.numpy as jnp
import numpy as np
from jax import lax
from jax.experimental import pallas as pl
from jax.experimental.pallas import tpu as pltpu

K, T, D, H = 128, 64, 128, 256
TEMP = 1.0
EPS = 1e-12
LOG2 = float(np.log(2.0))

KB = 32
GRID = K // KB

NEG_INF = np.float32(-np.inf)

_GUMBEL = np.asarray(
    jax.random.gumbel(jax.random.key(42), (K, T * T), jnp.float32)
)

_DIAGMASK = np.where(
    (np.arange(T * T) // T) == (np.arange(T * T) % T), -np.inf, 0.0
).astype(np.float32)[None, :]


def _body(emb_ref, gum_ref, dmask_ref, w1_ref, b1_ref, w2_ref, b2_ref,
          idx1_ref, idx2_ref, br1_ref, br2_ref, out_ref, logv_ref):
    diag2 = (lax.broadcasted_iota(jnp.int32, (T, T), 0)
             == lax.broadcasted_iota(jnp.int32, (T, T), 1))

    grams, ncols, nrows = [], [], []
    for i in range(KB):
        e = emb_ref[i]
        g = lax.dot_general(e, e, (((1,), (1,)), ((), ())),
                            preferred_element_type=jnp.float32,
                            precision=lax.Precision.HIGHEST)
        dv = jnp.where(diag2, g, 0.0)
        grams.append(g[None])
        ncols.append(jnp.sum(dv, axis=1, keepdims=True)[None])
        nrows.append(jnp.sum(dv, axis=0, keepdims=True)[None])

    g3 = jnp.concatenate(grams, axis=0)
    ncol3 = jnp.concatenate(ncols, axis=0)
    nrow3 = jnp.concatenate(nrows, axis=0)
    sq3 = jnp.maximum(ncol3 + nrow3 - 2.0 * g3, 0.0)
    sq2 = sq3.reshape(KB, T * T)
    lane = lax.broadcasted_iota(jnp.int32, (KB, T * T), 1)
    dist2 = jnp.sqrt(sq2 + EPS)
    logits2 = dmask_ref[:] - dist2 / TEMP
    scores2 = logits2 + gum_ref[:]

    m = jnp.max(scores2, axis=1, keepdims=True)
    samp = jnp.min(jnp.where(scores2 == m, lane, T * T),
                   axis=1, keepdims=True)
    i1s = samp // T
    i2s = samp % T
    chosen = jnp.sum(jnp.where(lane == samp, logits2, 0.0),
                     axis=1, keepdims=True)
    ml = jnp.max(logits2, axis=1, keepdims=True)
    s = jnp.sum(jnp.exp(logits2 - ml), axis=1, keepdims=True)
    lse = ml + jnp.log(s)
    idx1_ref[:] = i1s
    idx2_ref[:] = i2s
    logv_ref[:] = chosen + LOG2 - lse

    SB = 8
    kr = lax.broadcasted_iota(jnp.int32, (SB, 1), 0)
    ln = lax.broadcasted_iota(jnp.int32, (SB, SB * T), 1)
    c1_parts, c2_parts = [], []
    for sb in range(KB // SB):
        tgt1 = kr * T + i1s[sb * SB:(sb + 1) * SB, :]
        tgt2 = kr * T + i2s[sb * SB:(sb + 1) * SB, :]
        m1 = (ln == tgt1).astype(jnp.float32)
        m2 = (ln == tgt2).astype(jnp.float32)
        e_sb = emb_ref[sb * SB:(sb + 1) * SB].reshape(SB * T, D)
        c1_parts.append(lax.dot_general(
            m1, e_sb, (((1,), (0,)), ((), ())),
            preferred_element_type=jnp.float32,
            precision=lax.Precision.HIGHEST))
        c2_parts.append(lax.dot_general(
            m2, e_sb, (((1,), (0,)), ((), ())),
            preferred_element_type=jnp.float32,
            precision=lax.Precision.HIGHEST))
    c1 = jnp.concatenate(c1_parts, axis=0)
    c2 = jnp.concatenate(c2_parts, axis=0)
    cat = jnp.concatenate([c1, c2], axis=1)

    h = lax.dot_general(cat, w1_ref[:], (((1,), (0,)), ((), ())),
                        preferred_element_type=jnp.float32) + b1_ref[:]
    h = jnp.maximum(h, 0.0)
    out = lax.dot_general(h, w2_ref[:], (((1,), (0,)), ((), ())),
                          preferred_element_type=jnp.float32) + b2_ref[:]
    out_ref[:] = out
    br1_ref[:] = jnp.sqrt(jnp.sum((c1 - out) ** 2, axis=1, keepdims=True)
                          + EPS)
    br2_ref[:] = jnp.sqrt(jnp.sum((c2 - out) ** 2, axis=1, keepdims=True)
                          + EPS)


def kernel(N, leaf_counts_Kxt, embeddings_KxtxD, W1, b1, W2, b2):
    gum = jnp.asarray(_GUMBEL)
    idx1, idx2, br1, br2, emb_out, logv = pl.pallas_call(
        _body,
        grid=(GRID,),
        in_specs=[
            pl.BlockSpec((KB, T, D), lambda i: (i, 0, 0)),
            pl.BlockSpec((KB, T * T), lambda i: (i, 0)),
            pl.BlockSpec((1, T * T), lambda i: (0, 0)),
            pl.BlockSpec((2 * D, H), lambda i: (0, 0)),
            pl.BlockSpec((1, H), lambda i: (0, 0)),
            pl.BlockSpec((H, D), lambda i: (0, 0)),
            pl.BlockSpec((1, D), lambda i: (0, 0)),
        ],
        out_specs=[
            pl.BlockSpec((KB, 1), lambda i: (i, 0)),
            pl.BlockSpec((KB, 1), lambda i: (i, 0)),
            pl.BlockSpec((KB, 1), lambda i: (i, 0)),
            pl.BlockSpec((KB, 1), lambda i: (i, 0)),
            pl.BlockSpec((KB, D), lambda i: (i, 0)),
            pl.BlockSpec((KB, 1), lambda i: (i, 0)),
        ],
        out_shape=[
            jax.ShapeDtypeStruct((K, 1), jnp.int32),
            jax.ShapeDtypeStruct((K, 1), jnp.int32),
            jax.ShapeDtypeStruct((K, 1), jnp.float32),
            jax.ShapeDtypeStruct((K, 1), jnp.float32),
            jax.ShapeDtypeStruct((K, D), jnp.float32),
            jax.ShapeDtypeStruct((K, 1), jnp.float32),
        ],
    )(embeddings_KxtxD, gum, jnp.asarray(_DIAGMASK), W1,
      b1.reshape(1, H), W2, b2.reshape(1, D))
    return (idx1, idx2, br1, br2, emb_out[:, None, :], logv)


# speedup vs baseline: 5.3117x; 1.0121x over previous
import jax
import jax.numpy as jnp
import numpy as np
from jax import lax
from jax.experimental import pallas as pl
from jax.experimental.pallas import tpu as pltpu

K, T, D, H = 128, 64, 128, 256
TEMP = 1.0
EPS = 1e-12
LOG2 = float(np.log(2.0))

KB = 32
GRID = K // KB

NEG_INF = np.float32(-np.inf)

_GUMBEL = np.asarray(
    jax.random.gumbel(jax.random.key(42), (K, T * T), jnp.float32)
)

_DIAGMASK = np.where(
    (np.arange(T * T) // T) == (np.arange(T * T) % T), -np.inf, 0.0
).astype(np.float32)[None, :]

_LANEIDX = np.arange(T * T, dtype=np.int32)[None, :]


def _body(emb_ref, gum_ref, dmask_ref, lane_ref, w1_ref, b1_ref, w2_ref,
          b2_ref, idx1_ref, idx2_ref, br1_ref, br2_ref, out_ref, logv_ref):
    diag2 = (lax.broadcasted_iota(jnp.int32, (T, T), 0)
             == lax.broadcasted_iota(jnp.int32, (T, T), 1))

    grams, ncols, nrows = [], [], []
    for i in range(KB):
        e = emb_ref[i]
        g = lax.dot_general(e, e, (((1,), (1,)), ((), ())),
                            preferred_element_type=jnp.float32,
                            precision=lax.Precision.HIGHEST)
        dv = jnp.where(diag2, g, 0.0)
        grams.append(g[None])
        ncols.append(jnp.sum(dv, axis=1, keepdims=True)[None])
        nrows.append(jnp.sum(dv, axis=0, keepdims=True)[None])

    g3 = jnp.concatenate(grams, axis=0)
    ncol3 = jnp.concatenate(ncols, axis=0)
    nrow3 = jnp.concatenate(nrows, axis=0)
    sq3 = jnp.maximum(ncol3 + nrow3 - 2.0 * g3, 0.0)
    sq2 = sq3.reshape(KB, T * T)
    lane = lane_ref[:]
    dist2 = jnp.sqrt(sq2 + EPS)
    logits2 = dmask_ref[:] - dist2 / TEMP
    scores2 = logits2 + gum_ref[:]

    m = jnp.max(scores2, axis=1, keepdims=True)
    samp = jnp.min(jnp.where(scores2 == m, lane, T * T),
                   axis=1, keepdims=True)
    i1s = samp // T
    i2s = samp % T
    chosen = jnp.sum(jnp.where(lane == samp, logits2, 0.0),
                     axis=1, keepdims=True)
    ml = jnp.max(logits2, axis=1, keepdims=True)
    s = jnp.sum(jnp.exp(logits2 - ml), axis=1, keepdims=True)
    lse = ml + jnp.log(s)
    idx1_ref[:] = i1s
    idx2_ref[:] = i2s
    logv_ref[:] = chosen + LOG2 - lse

    SB = 4
    kr = lax.broadcasted_iota(jnp.int32, (2 * SB, 1), 0) % SB
    ln = lax.broadcasted_iota(jnp.int32, (2 * SB, SB * T), 1)
    c1_parts, c2_parts = [], []
    for sb in range(KB // SB):
        i12 = jnp.concatenate([i1s[sb * SB:(sb + 1) * SB, :],
                               i2s[sb * SB:(sb + 1) * SB, :]], axis=0)
        m12 = (ln == kr * T + i12).astype(jnp.float32)
        e_sb = emb_ref[sb * SB:(sb + 1) * SB].reshape(SB * T, D)
        c12 = lax.dot_general(
            m12, e_sb, (((1,), (0,)), ((), ())),
            preferred_element_type=jnp.float32,
            precision=lax.Precision.HIGHEST)
        c1_parts.append(c12[:SB])
        c2_parts.append(c12[SB:])
    c1 = jnp.concatenate(c1_parts, axis=0)
    c2 = jnp.concatenate(c2_parts, axis=0)
    cat = jnp.concatenate([c1, c2], axis=1)

    h = lax.dot_general(cat, w1_ref[:], (((1,), (0,)), ((), ())),
                        preferred_element_type=jnp.float32) + b1_ref[:]
    h = jnp.maximum(h, 0.0)
    out = lax.dot_general(h, w2_ref[:], (((1,), (0,)), ((), ())),
                          preferred_element_type=jnp.float32) + b2_ref[:]
    out_ref[:] = out
    br1_ref[:] = jnp.sqrt(jnp.sum((c1 - out) ** 2, axis=1, keepdims=True)
                          + EPS)
    br2_ref[:] = jnp.sqrt(jnp.sum((c2 - out) ** 2, axis=1, keepdims=True)
                          + EPS)


def kernel(N, leaf_counts_Kxt, embeddings_KxtxD, W1, b1, W2, b2):
    gum = jnp.asarray(_GUMBEL)
    idx1, idx2, br1, br2, emb_out, logv = pl.pallas_call(
        _body,
        grid=(GRID,),
        in_specs=[
            pl.BlockSpec((KB, T, D), lambda i: (i, 0, 0)),
            pl.BlockSpec((KB, T * T), lambda i: (i, 0)),
            pl.BlockSpec((1, T * T), lambda i: (0, 0)),
            pl.BlockSpec((1, T * T), lambda i: (0, 0)),
            pl.BlockSpec((2 * D, H), lambda i: (0, 0)),
            pl.BlockSpec((1, H), lambda i: (0, 0)),
            pl.BlockSpec((H, D), lambda i: (0, 0)),
            pl.BlockSpec((1, D), lambda i: (0, 0)),
        ],
        out_specs=[
            pl.BlockSpec((KB, 1), lambda i: (i, 0)),
            pl.BlockSpec((KB, 1), lambda i: (i, 0)),
            pl.BlockSpec((KB, 1), lambda i: (i, 0)),
            pl.BlockSpec((KB, 1), lambda i: (i, 0)),
            pl.BlockSpec((KB, D), lambda i: (i, 0)),
            pl.BlockSpec((KB, 1), lambda i: (i, 0)),
        ],
        out_shape=[
            jax.ShapeDtypeStruct((K, 1), jnp.int32),
            jax.ShapeDtypeStruct((K, 1), jnp.int32),
            jax.ShapeDtypeStruct((K, 1), jnp.float32),
            jax.ShapeDtypeStruct((K, 1), jnp.float32),
            jax.ShapeDtypeStruct((K, D), jnp.float32),
            jax.ShapeDtypeStruct((K, 1), jnp.float32),
        ],
    )(embeddings_KxtxD, gum, jnp.asarray(_DIAGMASK), jnp.asarray(_LANEIDX),
      W1, b1.reshape(1, H), W2, b2.reshape(1, D))
    return (idx1, idx2, br1, br2, emb_out[:, None, :], logv)


# speedup vs baseline: 5.7823x; 1.0886x over previous
import jax
import jax.numpy as jnp
import numpy as np
from jax import lax
from jax.experimental import pallas as pl
from jax.experimental.pallas import tpu as pltpu

K, T, D, H = 128, 64, 128, 256
TEMP = 1.0
EPS = 1e-12
LOG2 = float(np.log(2.0))

KB = 64
GRID = K // KB

NEG_INF = np.float32(-np.inf)

_GUMBEL = np.asarray(
    jax.random.gumbel(jax.random.key(42), (K, T * T), jnp.float32)
)

_DIAGMASK = np.where(
    (np.arange(T * T) // T) == (np.arange(T * T) % T), -np.inf, 0.0
).astype(np.float32)[None, :]

_LANEIDX = np.arange(T * T, dtype=np.int32)[None, :]


def _body(emb_ref, gum_ref, dmask_ref, lane_ref, w1_ref, b1_ref, w2_ref,
          b2_ref, idx1_ref, idx2_ref, br1_ref, br2_ref, out_ref, logv_ref):
    diag2 = (lax.broadcasted_iota(jnp.int32, (T, T), 0)
             == lax.broadcasted_iota(jnp.int32, (T, T), 1))

    grams, ncols, nrows = [], [], []
    for i in range(KB):
        e = emb_ref[i]
        g = lax.dot_general(e, e, (((1,), (1,)), ((), ())),
                            preferred_element_type=jnp.float32,
                            precision=lax.Precision.HIGHEST)
        dv = jnp.where(diag2, g, 0.0)
        grams.append(g[None])
        ncols.append(jnp.sum(dv, axis=1, keepdims=True)[None])
        nrows.append(jnp.sum(dv, axis=0, keepdims=True)[None])

    g3 = jnp.concatenate(grams, axis=0)
    ncol3 = jnp.concatenate(ncols, axis=0)
    nrow3 = jnp.concatenate(nrows, axis=0)
    sq3 = jnp.maximum(ncol3 + nrow3 - 2.0 * g3, 0.0)
    sq2 = sq3.reshape(KB, T * T)
    lane = lane_ref[:]
    dist2 = jnp.sqrt(sq2 + EPS)
    logits2 = dmask_ref[:] - dist2 / TEMP
    scores2 = logits2 + gum_ref[:]

    m = jnp.max(scores2, axis=1, keepdims=True)
    samp = jnp.min(jnp.where(scores2 == m, lane, T * T),
                   axis=1, keepdims=True)
    i1s = samp // T
    i2s = samp % T
    chosen = jnp.sum(jnp.where(lane == samp, logits2, 0.0),
                     axis=1, keepdims=True)
    ml = jnp.max(logits2, axis=1, keepdims=True)
    s = jnp.sum(jnp.exp(logits2 - ml), axis=1, keepdims=True)
    lse = ml + jnp.log(s)
    idx1_ref[:] = i1s
    idx2_ref[:] = i2s
    logv_ref[:] = chosen + LOG2 - lse

    SB = 4
    kr = lax.broadcasted_iota(jnp.int32, (2 * SB, 1), 0) % SB
    ln = lax.broadcasted_iota(jnp.int32, (2 * SB, SB * T), 1)
    c1_parts, c2_parts = [], []
    for sb in range(KB // SB):
        i12 = jnp.concatenate([i1s[sb * SB:(sb + 1) * SB, :],
                               i2s[sb * SB:(sb + 1) * SB, :]], axis=0)
        m12 = (ln == kr * T + i12).astype(jnp.float32)
        e_sb = emb_ref[sb * SB:(sb + 1) * SB].reshape(SB * T, D)
        c12 = lax.dot_general(
            m12, e_sb, (((1,), (0,)), ((), ())),
            preferred_element_type=jnp.float32,
            precision=lax.Precision.HIGHEST)
        c1_parts.append(c12[:SB])
        c2_parts.append(c12[SB:])
    c1 = jnp.concatenate(c1_parts, axis=0)
    c2 = jnp.concatenate(c2_parts, axis=0)
    cat = jnp.concatenate([c1, c2], axis=1)

    h = lax.dot_general(cat, w1_ref[:], (((1,), (0,)), ((), ())),
                        preferred_element_type=jnp.float32) + b1_ref[:]
    h = jnp.maximum(h, 0.0)
    out = lax.dot_general(h, w2_ref[:], (((1,), (0,)), ((), ())),
                          preferred_element_type=jnp.float32) + b2_ref[:]
    out_ref[:] = out
    br1_ref[:] = jnp.sqrt(jnp.sum((c1 - out) ** 2, axis=1, keepdims=True)
                          + EPS)
    br2_ref[:] = jnp.sqrt(jnp.sum((c2 - out) ** 2, axis=1, keepdims=True)
                          + EPS)


def kernel(N, leaf_counts_Kxt, embeddings_KxtxD, W1, b1, W2, b2):
    gum = jnp.asarray(_GUMBEL)
    idx1, idx2, br1, br2, emb_out, logv = pl.pallas_call(
        _body,
        grid=(GRID,),
        in_specs=[
            pl.BlockSpec((KB, T, D), lambda i: (i, 0, 0)),
            pl.BlockSpec((KB, T * T), lambda i: (i, 0)),
            pl.BlockSpec((1, T * T), lambda i: (0, 0)),
            pl.BlockSpec((1, T * T), lambda i: (0, 0)),
            pl.BlockSpec((2 * D, H), lambda i: (0, 0)),
            pl.BlockSpec((1, H), lambda i: (0, 0)),
            pl.BlockSpec((H, D), lambda i: (0, 0)),
            pl.BlockSpec((1, D), lambda i: (0, 0)),
        ],
        out_specs=[
            pl.BlockSpec((KB, 1), lambda i: (i, 0)),
            pl.BlockSpec((KB, 1), lambda i: (i, 0)),
            pl.BlockSpec((KB, 1), lambda i: (i, 0)),
            pl.BlockSpec((KB, 1), lambda i: (i, 0)),
            pl.BlockSpec((KB, D), lambda i: (i, 0)),
            pl.BlockSpec((KB, 1), lambda i: (i, 0)),
        ],
        out_shape=[
            jax.ShapeDtypeStruct((K, 1), jnp.int32),
            jax.ShapeDtypeStruct((K, 1), jnp.int32),
            jax.ShapeDtypeStruct((K, 1), jnp.float32),
            jax.ShapeDtypeStruct((K, 1), jnp.float32),
            jax.ShapeDtypeStruct((K, D), jnp.float32),
            jax.ShapeDtypeStruct((K, 1), jnp.float32),
        ],
    )(embeddings_KxtxD, gum, jnp.asarray(_DIAGMASK), jnp.asarray(_LANEIDX),
      W1, b1.reshape(1, H), W2, b2.reshape(1, D))
    return (idx1, idx2, br1, br2, emb_out[:, None, :], logv)


# speedup vs baseline: 5.8088x; 1.0046x over previous
import jax
import jax.numpy as jnp
import numpy as np
from jax import lax
from jax.experimental import pallas as pl
from jax.experimental.pallas import tpu as pltpu

K, T, D, H = 128, 64, 128, 256
TEMP = 1.0
EPS = 1e-12
LOG2 = float(np.log(2.0))

KB = 128
GRID = K // KB

NEG_INF = np.float32(-np.inf)

_GUMBEL = np.asarray(
    jax.random.gumbel(jax.random.key(42), (K, T * T), jnp.float32)
)

_DIAGMASK = np.where(
    (np.arange(T * T) // T) == (np.arange(T * T) % T), -np.inf, 0.0
).astype(np.float32)[None, :]

_LANEIDX = np.arange(T * T, dtype=np.int32)[None, :]


def _body(emb_ref, gum_ref, dmask_ref, lane_ref, w1_ref, b1_ref, w2_ref,
          b2_ref, idx1_ref, idx2_ref, br1_ref, br2_ref, out_ref, logv_ref):
    diag2 = (lax.broadcasted_iota(jnp.int32, (T, T), 0)
             == lax.broadcasted_iota(jnp.int32, (T, T), 1))

    grams, ncols, nrows = [], [], []
    for i in range(KB):
        e = emb_ref[i]
        g = lax.dot_general(e, e, (((1,), (1,)), ((), ())),
                            preferred_element_type=jnp.float32,
                            precision=lax.Precision.HIGHEST)
        dv = jnp.where(diag2, g, 0.0)
        grams.append(g[None])
        ncols.append(jnp.sum(dv, axis=1, keepdims=True)[None])
        nrows.append(jnp.sum(dv, axis=0, keepdims=True)[None])

    g3 = jnp.concatenate(grams, axis=0)
    ncol3 = jnp.concatenate(ncols, axis=0)
    nrow3 = jnp.concatenate(nrows, axis=0)
    sq3 = jnp.maximum(ncol3 + nrow3 - 2.0 * g3, 0.0)
    sq2 = sq3.reshape(KB, T * T)
    lane = lane_ref[:]
    dist2 = jnp.sqrt(sq2 + EPS)
    logits2 = dmask_ref[:] - dist2 / TEMP
    scores2 = logits2 + gum_ref[:]

    m = jnp.max(scores2, axis=1, keepdims=True)
    samp = jnp.min(jnp.where(scores2 == m, lane, T * T),
                   axis=1, keepdims=True)
    i1s = samp // T
    i2s = samp % T
    chosen = jnp.sum(jnp.where(lane == samp, logits2, 0.0),
                     axis=1, keepdims=True)
    ml = jnp.max(logits2, axis=1, keepdims=True)
    s = jnp.sum(jnp.exp(logits2 - ml), axis=1, keepdims=True)
    lse = ml + jnp.log(s)
    idx1_ref[:] = i1s
    idx2_ref[:] = i2s
    logv_ref[:] = chosen + LOG2 - lse

    SB = 4
    kr = lax.broadcasted_iota(jnp.int32, (2 * SB, 1), 0) % SB
    ln = lax.broadcasted_iota(jnp.int32, (2 * SB, SB * T), 1)
    c1_parts, c2_parts = [], []
    for sb in range(KB // SB):
        i12 = jnp.concatenate([i1s[sb * SB:(sb + 1) * SB, :],
                               i2s[sb * SB:(sb + 1) * SB, :]], axis=0)
        m12 = (ln == kr * T + i12).astype(jnp.float32)
        e_sb = emb_ref[sb * SB:(sb + 1) * SB].reshape(SB * T, D)
        c12 = lax.dot_general(
            m12, e_sb, (((1,), (0,)), ((), ())),
            preferred_element_type=jnp.float32,
            precision=lax.Precision.HIGHEST)
        c1_parts.append(c12[:SB])
        c2_parts.append(c12[SB:])
    c1 = jnp.concatenate(c1_parts, axis=0)
    c2 = jnp.concatenate(c2_parts, axis=0)
    cat = jnp.concatenate([c1, c2], axis=1)

    h = lax.dot_general(cat, w1_ref[:], (((1,), (0,)), ((), ())),
                        preferred_element_type=jnp.float32) + b1_ref[:]
    h = jnp.maximum(h, 0.0)
    out = lax.dot_general(h, w2_ref[:], (((1,), (0,)), ((), ())),
                          preferred_element_type=jnp.float32) + b2_ref[:]
    out_ref[:] = out
    br1_ref[:] = jnp.sqrt(jnp.sum((c1 - out) ** 2, axis=1, keepdims=True)
                          + EPS)
    br2_ref[:] = jnp.sqrt(jnp.sum((c2 - out) ** 2, axis=1, keepdims=True)
                          + EPS)


def kernel(N, leaf_counts_Kxt, embeddings_KxtxD, W1, b1, W2, b2):
    gum = jnp.asarray(_GUMBEL)
    idx1, idx2, br1, br2, emb_out, logv = pl.pallas_call(
        _body,
        grid=(GRID,),
        in_specs=[
            pl.BlockSpec((KB, T, D), lambda i: (i, 0, 0)),
            pl.BlockSpec((KB, T * T), lambda i: (i, 0)),
            pl.BlockSpec((1, T * T), lambda i: (0, 0)),
            pl.BlockSpec((1, T * T), lambda i: (0, 0)),
            pl.BlockSpec((2 * D, H), lambda i: (0, 0)),
            pl.BlockSpec((1, H), lambda i: (0, 0)),
            pl.BlockSpec((H, D), lambda i: (0, 0)),
            pl.BlockSpec((1, D), lambda i: (0, 0)),
        ],
        out_specs=[
            pl.BlockSpec((KB, 1), lambda i: (i, 0)),
            pl.BlockSpec((KB, 1), lambda i: (i, 0)),
            pl.BlockSpec((KB, 1), lambda i: (i, 0)),
            pl.BlockSpec((KB, 1), lambda i: (i, 0)),
            pl.BlockSpec((KB, D), lambda i: (i, 0)),
            pl.BlockSpec((KB, 1), lambda i: (i, 0)),
        ],
        out_shape=[
            jax.ShapeDtypeStruct((K, 1), jnp.int32),
            jax.ShapeDtypeStruct((K, 1), jnp.int32),
            jax.ShapeDtypeStruct((K, 1), jnp.float32),
            jax.ShapeDtypeStruct((K, 1), jnp.float32),
            jax.ShapeDtypeStruct((K, D), jnp.float32),
            jax.ShapeDtypeStruct((K, 1), jnp.float32),
        ],
    )(embeddings_KxtxD, gum, jnp.asarray(_DIAGMASK), jnp.asarray(_LANEIDX),
      W1, b1.reshape(1, H), W2, b2.reshape(1, D))
    return (idx1, idx2, br1, br2, emb_out[:, None, :], logv)
